# baseline (speedup 1.0000x reference)
import jax
import jax.numpy as jnp
from jax.experimental import pallas as pl
from jax.experimental.pallas import tpu as pltpu

_ROWS = 576
_COLS = 32768
_BLOCK_ROWS = 16


def _copy_body(x_ref, o_ref):
    o_ref[...] = x_ref[...]


def kernel(x):
    orig_shape = x.shape
    flat = x.reshape(_ROWS, _COLS)
    grid = (_ROWS // _BLOCK_ROWS,)
    out = pl.pallas_call(
        _copy_body,
        out_shape=jax.ShapeDtypeStruct(flat.shape, flat.dtype),
        grid=grid,
        in_specs=[pl.BlockSpec((_BLOCK_ROWS, _COLS), lambda i: (i, 0))],
        out_specs=pl.BlockSpec((_BLOCK_ROWS, _COLS), lambda i: (i, 0)),
        compiler_params=pltpu.CompilerParams(
            dimension_semantics=("parallel",),
        ),
    )(flat)
    return out.reshape(orig_shape)


# speedup vs baseline: 1.0214x; 1.0214x over previous
import jax
import jax.numpy as jnp
from jax.experimental import pallas as pl
from jax.experimental.pallas import tpu as pltpu

_ROWS = 576
_COLS = 32768
_BLOCK_ROWS = 16
_STEPS = _ROWS // _BLOCK_ROWS
_NBUF = 16
_LAG = 8


def _copy_body(x_ref, o_ref, buf, in_sems, out_sems):
    def in_copy(step, b):
        return pltpu.make_async_copy(
            x_ref.at[pl.ds(step * _BLOCK_ROWS, _BLOCK_ROWS)],
            buf.at[pl.ds(b * _BLOCK_ROWS, _BLOCK_ROWS)],
            in_sems.at[b],
        )

    def out_copy(step, b):
        return pltpu.make_async_copy(
            buf.at[pl.ds(b * _BLOCK_ROWS, _BLOCK_ROWS)],
            o_ref.at[pl.ds(step * _BLOCK_ROWS, _BLOCK_ROWS)],
            out_sems.at[b],
        )

    waited_out = set()
    for b in range(min(_NBUF, _STEPS)):
        in_copy(b, b).start()
    for s in range(_STEPS):
        in_copy(s, s % _NBUF).wait()
        out_copy(s, s % _NBUF).start()
        t = s - _LAG
        if t >= 0 and t + _NBUF < _STEPS:
            out_copy(t, t % _NBUF).wait()
            waited_out.add(t)
            in_copy(t + _NBUF, t % _NBUF).start()
    for t in range(_STEPS):
        if t not in waited_out:
            out_copy(t, t % _NBUF).wait()


def kernel(x):
    orig_shape = x.shape
    flat = x.reshape(_ROWS, _COLS)
    out = pl.pallas_call(
        _copy_body,
        out_shape=jax.ShapeDtypeStruct(flat.shape, flat.dtype),
        in_specs=[pl.BlockSpec(memory_space=pl.MemorySpace.ANY)],
        out_specs=pl.BlockSpec(memory_space=pl.MemorySpace.ANY),
        scratch_shapes=[
            pltpu.VMEM((_NBUF * _BLOCK_ROWS, _COLS), jnp.float32),
            pltpu.SemaphoreType.DMA((_NBUF,)),
            pltpu.SemaphoreType.DMA((_NBUF,)),
        ],
    )(flat)
    return out.reshape(orig_shape)


# speedup vs baseline: 4.5371x; 4.4421x over previous
import jax
import jax.numpy as jnp
from jax.experimental import pallas as pl
from jax.experimental.pallas import tpu as pltpu

_BLOCK_T = 8


def _copy_body(x_ref, o_ref):
    o_ref[...] = x_ref[...]


def kernel(x):
    n, t, d, h, w = x.shape
    out = pl.pallas_call(
        _copy_body,
        out_shape=jax.ShapeDtypeStruct(x.shape, x.dtype),
        grid=(t // _BLOCK_T,),
        in_specs=[
            pl.BlockSpec((n, _BLOCK_T, d, h, w), lambda i: (0, i, 0, 0, 0))
        ],
        out_specs=pl.BlockSpec(
            (n, _BLOCK_T, d, h, w), lambda i: (0, i, 0, 0, 0)
        ),
        compiler_params=pltpu.CompilerParams(
            dimension_semantics=("arbitrary",),
        ),
    )(x)
    return out


# speedup vs baseline: 4.6386x; 1.0224x over previous
import jax
import jax.numpy as jnp
from jax.experimental import pallas as pl
from jax.experimental.pallas import tpu as pltpu

_T = 128
_BLOCK_T = 8
_STEPS = _T // _BLOCK_T
_NBUF = 8
_LAG = 4


def _copy_body(x_ref, o_ref, buf, in_sems, out_sems):
    def in_copy(step, b):
        return pltpu.make_async_copy(
            x_ref.at[:, pl.ds(step * _BLOCK_T, _BLOCK_T)],
            buf.at[b],
            in_sems.at[b],
        )

    def out_copy(step, b):
        return pltpu.make_async_copy(
            buf.at[b],
            o_ref.at[:, pl.ds(step * _BLOCK_T, _BLOCK_T)],
            out_sems.at[b],
        )

    waited_out = set()
    for b in range(min(_NBUF, _STEPS)):
        in_copy(b, b).start()
    for s in range(_STEPS):
        in_copy(s, s % _NBUF).wait()
        out_copy(s, s % _NBUF).start()
        t = s - _LAG
        if t >= 0 and t + _NBUF < _STEPS:
            out_copy(t, t % _NBUF).wait()
            waited_out.add(t)
            in_copy(t + _NBUF, t % _NBUF).start()
    for t in range(_STEPS):
        if t not in waited_out:
            out_copy(t, t % _NBUF).wait()


def kernel(x):
    n, t, d, h, w = x.shape
    out = pl.pallas_call(
        _copy_body,
        out_shape=jax.ShapeDtypeStruct(x.shape, x.dtype),
        in_specs=[pl.BlockSpec(memory_space=pl.MemorySpace.ANY)],
        out_specs=pl.BlockSpec(memory_space=pl.MemorySpace.ANY),
        scratch_shapes=[
            pltpu.VMEM((_NBUF, n, _BLOCK_T, d, h, w), jnp.float32),
            pltpu.SemaphoreType.DMA((_NBUF,)),
            pltpu.SemaphoreType.DMA((_NBUF,)),
        ],
    )(x)
    return out
